# Initial kernel scaffold; baseline (speedup 1.0000x reference)
import jax
import jax.numpy as jnp
from jax.experimental import pallas as pl


def kernel(gene_feat, emb_drug, Wf, bf, W_et, b_et, W_self, b_self, drug_identity, edge_dd, edge_dg, edge_gd, edge_gg):
    raise NotImplementedError("write your pallas kernel here")


# speedup vs baseline: 2.1473x; 2.1473x over previous
import functools

import jax
import jax.numpy as jnp
from jax import lax
from jax.experimental import pallas as pl
from jax.experimental.pallas import tpu as pltpu
from jax.experimental.pallas import tpu_sc as plsc

N_DRUG = 100000
N_GENE = 50000
E = 150000
D = 128
L = 2

NPD = 100352
NPG = 50176
EP = 153600
EROWS = EP // 128
TROWS = EROWS // 16
NBUF = 3
NBATCH = TROWS // NBUF
ZR = 196
RPT_D = NPD // 16
RPT_G = NPG // 16
BM = 1024


_SC_PARAMS = pltpu.CompilerParams(use_tc_tiling_on_sc=False)
_MESH = plsc.VectorSubcoreMesh(core_axis_name="c", subcore_axis_name="s")


def _sc_counts_build():
  f32 = jnp.float32
  out_type = [
      jax.ShapeDtypeStruct((NPD,), f32),
      jax.ShapeDtypeStruct((NPD,), f32),
      jax.ShapeDtypeStruct((NPG,), f32),
      jax.ShapeDtypeStruct((NPG,), f32),
  ]
  scratch_types = [
      pltpu.VMEM_SHARED((NPD,), f32),
      pltpu.VMEM((TROWS, 128), jnp.int32),
      pltpu.VMEM((ZR * 16,), f32),
      pltpu.VMEM((128,), f32),
  ]

  @functools.partial(pl.kernel, mesh=_MESH, out_type=out_type,
                     scratch_types=scratch_types, compiler_params=_SC_PARAMS)
  def sc_counts(d_dd, d_gd, d_dg, d_gg,
                cnt_dd, cnt_gd, cnt_dg, cnt_gg,
                cntacc, dstidx, zcnt, ones):
    c = lax.axis_index("c")
    s = lax.axis_index("s")

    def zfill1(i, _):
      zcnt[pl.ds(i * 16, 16)] = jnp.zeros((16,), f32)
      return 0
    lax.fori_loop(0, ZR, zfill1, 0)
    def ofill(i, _):
      ones[pl.ds(i * 16, 16)] = jnp.ones((16,), f32)
      return 0
    lax.fori_loop(0, 8, ofill, 0)

    def do_counts(dst2d, cnt_out, rpt, nz):
      pltpu.sync_copy(dst2d.at[pl.ds(s * TROWS, TROWS)], dstidx)
      for k in range(nz):
        pltpu.sync_copy(zcnt, cntacc.at[pl.ds(s * rpt + k * ZR * 16, ZR * 16)])
      plsc.subcore_barrier()
      def cbody(g, _):
        pltpu.sync_copy(ones, cntacc.at[dstidx.at[g]], add=True)
        return 0
      lax.fori_loop(0, TROWS, cbody, 0)
      plsc.subcore_barrier()
      pltpu.sync_copy(cntacc.at[pl.ds(s * rpt, rpt)],
                      cnt_out.at[pl.ds(s * rpt, rpt)])
      plsc.subcore_barrier()

    @pl.when(c == 0)
    def _():
      do_counts(d_dd, cnt_dd, RPT_D, 2)
      do_counts(d_gd, cnt_gd, RPT_D, 2)
    @pl.when(c == 1)
    def _():
      do_counts(d_dg, cnt_dg, RPT_G, 1)
      do_counts(d_gg, cnt_gg, RPT_G, 1)

  return sc_counts


def _sc_agg_build():
  f32 = jnp.float32
  out_type = [
      jax.ShapeDtypeStruct((NPD, 128), f32),
      jax.ShapeDtypeStruct((NPD, 128), f32),
      jax.ShapeDtypeStruct((NPG, 128), f32),
      jax.ShapeDtypeStruct((NPG, 128), f32),
  ]
  scratch_types = [
      pltpu.VMEM_SHARED((NPD, 16), f32),
      pltpu.VMEM((TROWS, 128), jnp.int32),
      pltpu.VMEM((TROWS, 128), jnp.int32),
      pltpu.VMEM((NBUF, 128, 16), f32),
      pltpu.VMEM((ZR, 16), f32),
  ] + [pltpu.SemaphoreType.DMA] * NBUF

  @functools.partial(pl.kernel, mesh=_MESH, out_type=out_type,
                     scratch_types=scratch_types, compiler_params=_SC_PARAMS)
  def sc_agg(
             hd0, hd1, hd2, hd3, hd4, hd5, hd6, hd7,
             hg0, hg1, hg2, hg3, hg4, hg5, hg6, hg7,
             s_dd, d_dd, s_dg, d_dg, s_gd, d_gd, s_gg, d_gg,
             agg_dd, agg_gd, agg_dg, agg_gg,
             acc, srcidx, dstidx, rowbuf, zrow,
             sem0, sem1, sem2):
    c = lax.axis_index("c")
    s = lax.axis_index("s")
    sems = [sem0, sem1, sem2]
    hd = [hd0, hd1, hd2, hd3, hd4, hd5, hd6, hd7]
    hg = [hg0, hg1, hg2, hg3, hg4, hg5, hg6, hg7]

    def zfill(i, _):
      zrow[i] = jnp.zeros((16,), f32)
      return 0
    lax.fori_loop(0, ZR, zfill, 0)

    def load_edges(src2d, dst2d):
      base = s * TROWS
      pltpu.sync_copy(src2d.at[pl.ds(base, TROWS)], srcidx)
      pltpu.sync_copy(dst2d.at[pl.ds(base, TROWS)], dstidx)

    def do_chunk(hsplit, agg_out, p, rpt, nz):
      for k in range(nz):
        pltpu.sync_copy(zrow, acc.at[pl.ds(s * rpt + k * ZR, ZR)])
      plsc.subcore_barrier()
      def bbody(bi, _):
        cps = []
        for b in range(NBUF):
          g = bi * NBUF + b
          cps.append(pltpu.async_copy(hsplit.at[srcidx.at[g]],
                                      rowbuf.at[b], sems[b]))
        for b in range(NBUF):
          cps[b].wait()
          g = bi * NBUF + b
          pltpu.sync_copy(rowbuf.at[b], acc.at[dstidx.at[g]], add=True)
        return 0
      lax.fori_loop(0, NBATCH, bbody, 0)
      plsc.subcore_barrier()
      pltpu.sync_copy(acc.at[pl.ds(s * rpt, rpt)],
                      agg_out.at[pl.ds(s * rpt, rpt), pl.ds(p * 16, 16)])
      plsc.subcore_barrier()

    etys = [
        (s_dd, d_dd, hd, agg_dd, RPT_D, 32),
        (s_gd, d_gd, hg, agg_gd, RPT_D, 32),
        (s_dg, d_dg, hd, agg_dg, RPT_G, 16),
        (s_gg, d_gg, hg, agg_gg, RPT_G, 16),
    ]
    for cc in range(2):
      @pl.when(c == cc)
      def _():
        for (src2d, dst2d, hs, agg_out, rpt, nz) in etys:
          load_edges(src2d, dst2d)
          for j in range(4):
            do_chunk(hs[cc * 4 + j], agg_out, cc * 4 + j, rpt, nz)

  return sc_agg


_sc_counts = _sc_counts_build()
_sc_agg = _sc_agg_build()


def _proj(x, W, b):
  n = x.shape[0]
  def body(x_ref, w_ref, b_ref, o_ref):
    o_ref[:] = (jnp.dot(x_ref[:], w_ref[:], preferred_element_type=jnp.float32,
                  precision=lax.Precision.HIGHEST)
                + b_ref[:][None, :])
  return pl.pallas_call(
      body,
      grid=(n // BM,),
      in_specs=[
          pl.BlockSpec((BM, 128), lambda i: (i, 0)),
          pl.BlockSpec((128, 128), lambda i: (0, 0)),
          pl.BlockSpec((128,), lambda i: (0,)),
      ],
      out_specs=pl.BlockSpec((BM, 128), lambda i: (i, 0)),
      out_shape=jax.ShapeDtypeStruct((n, 128), jnp.float32),
  )(x, W, b)


def _layer_nt(agg_a, cnt_a, Wa, ba, agg_b, cnt_b, Wb, bb, h, Ws, bs):
  n = h.shape[0]
  def body(aa_ref, ab_ref, h_ref, ca_ref, cb_ref,
           wa_ref, wb_ref, ws_ref, ba_ref, bb_ref, bs_ref, o_ref):
    ca = ca_ref[:]
    cb = cb_ref[:]
    ia = 1.0 / jnp.maximum(ca, 1.0)
    ib = 1.0 / jnp.maximum(cb, 1.0)
    ma = (ca > 0.0).astype(jnp.float32)
    mb = (cb > 0.0).astype(jnp.float32)
    xa = aa_ref[:] * ia[:, None]
    xb = ab_ref[:] * ib[:, None]
    na = (jnp.dot(xa, wa_ref[:], preferred_element_type=jnp.float32,
                  precision=lax.Precision.HIGHEST)
          + ma[:, None] * ba_ref[:][None, :])
    nb = (jnp.dot(xb, wb_ref[:], preferred_element_type=jnp.float32,
                  precision=lax.Precision.HIGHEST)
          + mb[:, None] * bb_ref[:][None, :])
    hs = (jnp.dot(h_ref[:], ws_ref[:], preferred_element_type=jnp.float32,
                  precision=lax.Precision.HIGHEST)
          + bs_ref[:][None, :])
    o_ref[:] = jnp.maximum(0.5 * (na + nb) + hs, 0.0)
  mat = lambda: pl.BlockSpec((BM, 128), lambda i: (i, 0))
  vec = lambda: pl.BlockSpec((BM,), lambda i: (i,))
  wsp = lambda: pl.BlockSpec((128, 128), lambda i: (0, 0))
  bsp = lambda: pl.BlockSpec((128,), lambda i: (0,))
  return pl.pallas_call(
      body,
      grid=(n // BM,),
      in_specs=[mat(), mat(), mat(), vec(), vec(),
                wsp(), wsp(), wsp(), bsp(), bsp(), bsp()],
      out_specs=pl.BlockSpec((BM, 128), lambda i: (i, 0)),
      out_shape=jax.ShapeDtypeStruct((n, 128), jnp.float32),
  )(agg_a, agg_b, h, cnt_a, cnt_b, Wa, Wb, Ws, ba, bb, bs)


def kernel(gene_feat, emb_drug, Wf, bf, W_et, b_et, W_self, b_self,
           drug_identity, edge_dd, edge_dg, edge_gd, edge_gg):
  f32 = jnp.float32
  i32 = jnp.int32

  h_d = jnp.pad(emb_drug, ((0, NPD - N_DRUG), (0, 0)))
  gene_p = jnp.pad(gene_feat, ((0, NPG - N_GENE), (0, 0)))

  padn = EP - E
  ar = jnp.arange(padn, dtype=i32)
  pad_src_d = ar % N_DRUG
  pad_src_g = ar % N_GENE
  pad_dst_d = N_DRUG + ar % (NPD - N_DRUG)
  pad_dst_g = N_GENE + ar % (NPG - N_GENE)

  def prep(e, src_is_drug, dst_is_drug):
    src = jnp.concatenate([e[0], pad_src_d if src_is_drug else pad_src_g])
    dst = jnp.concatenate([e[1], pad_dst_d if dst_is_drug else pad_dst_g])
    return src.reshape(EROWS, 128), dst.reshape(EROWS, 128)

  s_dd, d_dd = prep(edge_dd, True, True)
  s_dg, d_dg = prep(edge_dg, True, False)
  s_gd, d_gd = prep(edge_gd, False, True)
  s_gg, d_gg = prep(edge_gg, False, False)

  h_g = _proj(gene_p, Wf, bf)

  cnt_dd, cnt_gd, cnt_dg, cnt_gg = _sc_counts(d_dd, d_gd, d_dg, d_gg)

  h = [h_d, h_g]
  for l in range(L):
    hd_splits = [lax.slice(h[0], (0, 16 * p), (NPD, 16 * (p + 1)))
                 for p in range(8)]
    hg_splits = [lax.slice(h[1], (0, 16 * p), (NPG, 16 * (p + 1)))
                 for p in range(8)]
    agg_dd, agg_gd, agg_dg, agg_gg = _sc_agg(
        *hd_splits, *hg_splits,
        s_dd, d_dd, s_dg, d_dg, s_gd, d_gd, s_gg, d_gg)
    new_hd = _layer_nt(agg_dd, cnt_dd, W_et[l, 0], b_et[l, 0],
                       agg_gd, cnt_gd, W_et[l, 2], b_et[l, 2],
                       h[0], W_self[l, 0], b_self[l, 0])
    new_hg = _layer_nt(agg_dg, cnt_dg, W_et[l, 1], b_et[l, 1],
                       agg_gg, cnt_gg, W_et[l, 3], b_et[l, 3],
                       h[1], W_self[l, 1], b_self[l, 1])
    h = [new_hd, new_hg]

  return (h[0][:N_DRUG], h[1][:N_GENE])


# speedup vs baseline: 2.2747x; 1.0593x over previous
import functools

import jax
import jax.numpy as jnp
from jax import lax
from jax.experimental import pallas as pl
from jax.experimental.pallas import tpu as pltpu
from jax.experimental.pallas import tpu_sc as plsc

N_DRUG = 100000
N_GENE = 50000
E = 150000
D = 128
L = 2

NPD = 100352
NPG = 50176
EP = 153600
EROWS = EP // 128
TROWS = EROWS // 16
NBUF = 3
NBATCH = TROWS // NBUF
ZR = 196
RPT_D = NPD // 16
RPT_G = NPG // 16
BM = 1024


_SC_PARAMS = pltpu.CompilerParams(use_tc_tiling_on_sc=False)
_MESH = plsc.VectorSubcoreMesh(core_axis_name="c", subcore_axis_name="s")


def _sc_counts_build():
  f32 = jnp.float32
  out_type = [
      jax.ShapeDtypeStruct((NPD,), f32),
      jax.ShapeDtypeStruct((NPD,), f32),
      jax.ShapeDtypeStruct((NPG,), f32),
      jax.ShapeDtypeStruct((NPG,), f32),
  ]
  scratch_types = [
      pltpu.VMEM_SHARED((NPD,), f32),
      pltpu.VMEM((TROWS, 128), jnp.int32),
      pltpu.VMEM((ZR * 16,), f32),
      pltpu.VMEM((128,), f32),
  ] + [pltpu.SemaphoreType.DMA] * NBUF

  @functools.partial(pl.kernel, mesh=_MESH, out_type=out_type,
                     scratch_types=scratch_types, compiler_params=_SC_PARAMS)
  def sc_counts(d_dd, d_gd, d_dg, d_gg,
                cnt_dd, cnt_gd, cnt_dg, cnt_gg,
                cntacc, dstidx, zcnt, ones, sem0, sem1, sem2):
    c = lax.axis_index("c")
    s = lax.axis_index("s")
    sems = [sem0, sem1, sem2]

    def zfill1(i, _):
      zcnt[pl.ds(i * 16, 16)] = jnp.zeros((16,), f32)
      return 0
    lax.fori_loop(0, ZR, zfill1, 0)
    def ofill(i, _):
      ones[pl.ds(i * 16, 16)] = jnp.ones((16,), f32)
      return 0
    lax.fori_loop(0, 8, ofill, 0)

    def do_counts(dst2d, cnt_out, rpt, nz):
      pltpu.sync_copy(dst2d.at[pl.ds(s * TROWS, TROWS)], dstidx)
      for k in range(nz):
        pltpu.sync_copy(zcnt, cntacc.at[pl.ds(s * rpt + k * ZR * 16, ZR * 16)])
      plsc.subcore_barrier()
      def cfire(g, b):
        pltpu.async_copy(ones, cntacc.at[dstidx.at[g]], sems[b], add=True)
      def cdrain(g, b):
        pltpu.make_async_copy(ones, cntacc.at[dstidx.at[g]], sems[b]).wait()
      for b in range(NBUF):
        cfire(b, b)
      def cbody(bi, _):
        for b in range(NBUF):
          g = bi * NBUF + b
          cdrain(g, b)
          cfire(g + NBUF, b)
        return 0
      lax.fori_loop(0, NBATCH - 1, cbody, 0)
      for b in range(NBUF):
        cdrain((NBATCH - 1) * NBUF + b, b)
      plsc.subcore_barrier()
      pltpu.sync_copy(cntacc.at[pl.ds(s * rpt, rpt)],
                      cnt_out.at[pl.ds(s * rpt, rpt)])
      plsc.subcore_barrier()

    @pl.when(c == 0)
    def _():
      do_counts(d_dd, cnt_dd, RPT_D, 2)
      do_counts(d_gd, cnt_gd, RPT_D, 2)
    @pl.when(c == 1)
    def _():
      do_counts(d_dg, cnt_dg, RPT_G, 1)
      do_counts(d_gg, cnt_gg, RPT_G, 1)

  return sc_counts


def _sc_agg_build():
  f32 = jnp.float32
  out_type = [
      jax.ShapeDtypeStruct((NPD, 128), f32),
      jax.ShapeDtypeStruct((NPD, 128), f32),
      jax.ShapeDtypeStruct((NPG, 128), f32),
      jax.ShapeDtypeStruct((NPG, 128), f32),
  ]
  scratch_types = [
      pltpu.VMEM_SHARED((NPD, 16), f32),
      pltpu.VMEM((TROWS, 128), jnp.int32),
      pltpu.VMEM((TROWS, 128), jnp.int32),
      pltpu.VMEM((NBUF, 128, 16), f32),
      pltpu.VMEM((ZR, 16), f32),
  ] + [pltpu.SemaphoreType.DMA] * NBUF

  @functools.partial(pl.kernel, mesh=_MESH, out_type=out_type,
                     scratch_types=scratch_types, compiler_params=_SC_PARAMS)
  def sc_agg(
             hd0, hd1, hd2, hd3, hd4, hd5, hd6, hd7,
             hg0, hg1, hg2, hg3, hg4, hg5, hg6, hg7,
             s_dd, d_dd, s_dg, d_dg, s_gd, d_gd, s_gg, d_gg,
             agg_dd, agg_gd, agg_dg, agg_gg,
             acc, srcidx, dstidx, rowbuf, zrow,
             sem0, sem1, sem2):
    c = lax.axis_index("c")
    s = lax.axis_index("s")
    sems = [sem0, sem1, sem2]
    hd = [hd0, hd1, hd2, hd3, hd4, hd5, hd6, hd7]
    hg = [hg0, hg1, hg2, hg3, hg4, hg5, hg6, hg7]

    def zfill(i, _):
      zrow[i] = jnp.zeros((16,), f32)
      return 0
    lax.fori_loop(0, ZR, zfill, 0)

    def load_edges(src2d, dst2d):
      base = s * TROWS
      pltpu.sync_copy(src2d.at[pl.ds(base, TROWS)], srcidx)
      pltpu.sync_copy(dst2d.at[pl.ds(base, TROWS)], dstidx)

    def do_chunk(hsplit, agg_out, p, rpt, nz):
      for k in range(nz):
        pltpu.sync_copy(zrow, acc.at[pl.ds(s * rpt + k * ZR, ZR)])
      plsc.subcore_barrier()
      def fire(g, b):
        pltpu.async_copy(hsplit.at[srcidx.at[g]], rowbuf.at[b], sems[b])
      def drain_scatter(g, b):
        pltpu.make_async_copy(hsplit.at[srcidx.at[g]],
                              rowbuf.at[b], sems[b]).wait()
        pltpu.sync_copy(rowbuf.at[b], acc.at[dstidx.at[g]], add=True)
      for b in range(NBUF):
        fire(b, b)
      def bbody(bi, _):
        for b in range(NBUF):
          g = bi * NBUF + b
          drain_scatter(g, b)
          fire(g + NBUF, b)
        return 0
      lax.fori_loop(0, NBATCH - 1, bbody, 0)
      for b in range(NBUF):
        drain_scatter((NBATCH - 1) * NBUF + b, b)
      plsc.subcore_barrier()
      pltpu.sync_copy(acc.at[pl.ds(s * rpt, rpt)],
                      agg_out.at[pl.ds(s * rpt, rpt), pl.ds(p * 16, 16)])
      plsc.subcore_barrier()

    etys = [
        (s_dd, d_dd, hd, agg_dd, RPT_D, 32),
        (s_gd, d_gd, hg, agg_gd, RPT_D, 32),
        (s_dg, d_dg, hd, agg_dg, RPT_G, 16),
        (s_gg, d_gg, hg, agg_gg, RPT_G, 16),
    ]
    for cc in range(2):
      @pl.when(c == cc)
      def _():
        for (src2d, dst2d, hs, agg_out, rpt, nz) in etys:
          load_edges(src2d, dst2d)
          for j in range(4):
            do_chunk(hs[cc * 4 + j], agg_out, cc * 4 + j, rpt, nz)

  return sc_agg


_sc_counts = _sc_counts_build()
_sc_agg = _sc_agg_build()


def _proj(x, W, b):
  n = x.shape[0]
  def body(x_ref, w_ref, b_ref, o_ref):
    o_ref[:] = (jnp.dot(x_ref[:], w_ref[:], preferred_element_type=jnp.float32,
                  precision=lax.Precision.HIGHEST)
                + b_ref[:][None, :])
  return pl.pallas_call(
      body,
      grid=(n // BM,),
      in_specs=[
          pl.BlockSpec((BM, 128), lambda i: (i, 0)),
          pl.BlockSpec((128, 128), lambda i: (0, 0)),
          pl.BlockSpec((128,), lambda i: (0,)),
      ],
      out_specs=pl.BlockSpec((BM, 128), lambda i: (i, 0)),
      out_shape=jax.ShapeDtypeStruct((n, 128), jnp.float32),
  )(x, W, b)


def _layer_nt(agg_a, cnt_a, Wa, ba, agg_b, cnt_b, Wb, bb, h, Ws, bs):
  n = h.shape[0]
  def body(aa_ref, ab_ref, h_ref, ca_ref, cb_ref,
           wa_ref, wb_ref, ws_ref, ba_ref, bb_ref, bs_ref, o_ref):
    ca = ca_ref[:]
    cb = cb_ref[:]
    ia = 1.0 / jnp.maximum(ca, 1.0)
    ib = 1.0 / jnp.maximum(cb, 1.0)
    ma = (ca > 0.0).astype(jnp.float32)
    mb = (cb > 0.0).astype(jnp.float32)
    xa = aa_ref[:] * ia[:, None]
    xb = ab_ref[:] * ib[:, None]
    na = (jnp.dot(xa, wa_ref[:], preferred_element_type=jnp.float32,
                  precision=lax.Precision.HIGHEST)
          + ma[:, None] * ba_ref[:][None, :])
    nb = (jnp.dot(xb, wb_ref[:], preferred_element_type=jnp.float32,
                  precision=lax.Precision.HIGHEST)
          + mb[:, None] * bb_ref[:][None, :])
    hs = (jnp.dot(h_ref[:], ws_ref[:], preferred_element_type=jnp.float32,
                  precision=lax.Precision.HIGHEST)
          + bs_ref[:][None, :])
    o_ref[:] = jnp.maximum(0.5 * (na + nb) + hs, 0.0)
  mat = lambda: pl.BlockSpec((BM, 128), lambda i: (i, 0))
  vec = lambda: pl.BlockSpec((BM,), lambda i: (i,))
  wsp = lambda: pl.BlockSpec((128, 128), lambda i: (0, 0))
  bsp = lambda: pl.BlockSpec((128,), lambda i: (0,))
  return pl.pallas_call(
      body,
      grid=(n // BM,),
      in_specs=[mat(), mat(), mat(), vec(), vec(),
                wsp(), wsp(), wsp(), bsp(), bsp(), bsp()],
      out_specs=pl.BlockSpec((BM, 128), lambda i: (i, 0)),
      out_shape=jax.ShapeDtypeStruct((n, 128), jnp.float32),
  )(agg_a, agg_b, h, cnt_a, cnt_b, Wa, Wb, Ws, ba, bb, bs)


def kernel(gene_feat, emb_drug, Wf, bf, W_et, b_et, W_self, b_self,
           drug_identity, edge_dd, edge_dg, edge_gd, edge_gg):
  f32 = jnp.float32
  i32 = jnp.int32

  h_d = jnp.pad(emb_drug, ((0, NPD - N_DRUG), (0, 0)))
  gene_p = jnp.pad(gene_feat, ((0, NPG - N_GENE), (0, 0)))

  padn = EP - E
  ar = jnp.arange(padn, dtype=i32)
  pad_src_d = ar % N_DRUG
  pad_src_g = ar % N_GENE
  pad_dst_d = N_DRUG + ar % (NPD - N_DRUG)
  pad_dst_g = N_GENE + ar % (NPG - N_GENE)

  def prep(e, src_is_drug, dst_is_drug):
    src = jnp.concatenate([e[0], pad_src_d if src_is_drug else pad_src_g])
    dst = jnp.concatenate([e[1], pad_dst_d if dst_is_drug else pad_dst_g])
    return src.reshape(EROWS, 128), dst.reshape(EROWS, 128)

  s_dd, d_dd = prep(edge_dd, True, True)
  s_dg, d_dg = prep(edge_dg, True, False)
  s_gd, d_gd = prep(edge_gd, False, True)
  s_gg, d_gg = prep(edge_gg, False, False)

  h_g = _proj(gene_p, Wf, bf)

  cnt_dd, cnt_gd, cnt_dg, cnt_gg = _sc_counts(d_dd, d_gd, d_dg, d_gg)

  h = [h_d, h_g]
  for l in range(L):
    hd_splits = [lax.slice(h[0], (0, 16 * p), (NPD, 16 * (p + 1)))
                 for p in range(8)]
    hg_splits = [lax.slice(h[1], (0, 16 * p), (NPG, 16 * (p + 1)))
                 for p in range(8)]
    agg_dd, agg_gd, agg_dg, agg_gg = _sc_agg(
        *hd_splits, *hg_splits,
        s_dd, d_dd, s_dg, d_dg, s_gd, d_gd, s_gg, d_gg)
    new_hd = _layer_nt(agg_dd, cnt_dd, W_et[l, 0], b_et[l, 0],
                       agg_gd, cnt_gd, W_et[l, 2], b_et[l, 2],
                       h[0], W_self[l, 0], b_self[l, 0])
    new_hg = _layer_nt(agg_dg, cnt_dg, W_et[l, 1], b_et[l, 1],
                       agg_gg, cnt_gg, W_et[l, 3], b_et[l, 3],
                       h[1], W_self[l, 1], b_self[l, 1])
    h = [new_hd, new_hg]

  return (h[0][:N_DRUG], h[1][:N_GENE])


# speedup vs baseline: 3.4450x; 1.5145x over previous
import functools

import jax
import jax.numpy as jnp
from jax import lax
from jax.experimental import pallas as pl
from jax.experimental.pallas import tpu as pltpu
from jax.experimental.pallas import tpu_sc as plsc

N_DRUG = 100000
N_GENE = 50000
E = 150000
D = 128
L = 2

NPD = 100352
NPG = 50176
EP = 153600
EROWS = EP // 128
TROWS = EROWS // 16
NBUF = 3
NBATCH = TROWS // NBUF
ZR = 196
RPT_D = NPD // 16
RPT_G = NPG // 16
BM = 1024


_SC_PARAMS = pltpu.CompilerParams(use_tc_tiling_on_sc=False)
_MESH = plsc.VectorSubcoreMesh(core_axis_name="c", subcore_axis_name="s")


def _sc_counts_build():
  f32 = jnp.float32
  out_type = [
      jax.ShapeDtypeStruct((NPD,), f32),
      jax.ShapeDtypeStruct((NPD,), f32),
      jax.ShapeDtypeStruct((NPG,), f32),
      jax.ShapeDtypeStruct((NPG,), f32),
  ]
  scratch_types = [
      pltpu.VMEM_SHARED((NPD,), f32),
      pltpu.VMEM((TROWS, 128), jnp.int32),
      pltpu.VMEM((ZR * 16,), f32),
      pltpu.VMEM((128,), f32),
  ] + [pltpu.SemaphoreType.DMA] * NBUF

  @functools.partial(pl.kernel, mesh=_MESH, out_type=out_type,
                     scratch_types=scratch_types, compiler_params=_SC_PARAMS)
  def sc_counts(d_dd, d_gd, d_dg, d_gg,
                cnt_dd, cnt_gd, cnt_dg, cnt_gg,
                cntacc, dstidx, zcnt, ones, sem0, sem1, sem2):
    c = lax.axis_index("c")
    s = lax.axis_index("s")
    sems = [sem0, sem1, sem2]

    def zfill1(i, _):
      zcnt[pl.ds(i * 16, 16)] = jnp.zeros((16,), f32)
      return 0
    lax.fori_loop(0, ZR, zfill1, 0)
    def ofill(i, _):
      ones[pl.ds(i * 16, 16)] = jnp.ones((16,), f32)
      return 0
    lax.fori_loop(0, 8, ofill, 0)

    def do_counts(dst2d, cnt_out, rpt, nz):
      pltpu.sync_copy(dst2d.at[pl.ds(s * TROWS, TROWS)], dstidx)
      for k in range(nz):
        pltpu.sync_copy(zcnt, cntacc.at[pl.ds(s * rpt + k * ZR * 16, ZR * 16)])
      plsc.subcore_barrier()
      def cfire(g, b):
        pltpu.async_copy(ones, cntacc.at[dstidx.at[g]], sems[b], add=True)
      def cdrain(g, b):
        pltpu.make_async_copy(ones, cntacc.at[dstidx.at[g]], sems[b]).wait()
      for b in range(NBUF):
        cfire(b, b)
      def cbody(bi, _):
        for b in range(NBUF):
          g = bi * NBUF + b
          cdrain(g, b)
          cfire(g + NBUF, b)
        return 0
      lax.fori_loop(0, NBATCH - 1, cbody, 0)
      for b in range(NBUF):
        cdrain((NBATCH - 1) * NBUF + b, b)
      plsc.subcore_barrier()
      pltpu.sync_copy(cntacc.at[pl.ds(s * rpt, rpt)],
                      cnt_out.at[pl.ds(s * rpt, rpt)])
      plsc.subcore_barrier()

    @pl.when(c == 0)
    def _():
      do_counts(d_dd, cnt_dd, RPT_D, 2)
      do_counts(d_gd, cnt_gd, RPT_D, 2)
    @pl.when(c == 1)
    def _():
      do_counts(d_dg, cnt_dg, RPT_G, 1)
      do_counts(d_gg, cnt_gg, RPT_G, 1)

  return sc_counts


SPB = 1568


def _sc_split_build():
  f32 = jnp.float32
  out_type = [
      jax.ShapeDtypeStruct((8, NPD, 16), f32),
      jax.ShapeDtypeStruct((8, NPG, 16), f32),
  ]
  scratch_types = [pltpu.VMEM((SPB, 16), f32)]

  @functools.partial(pl.kernel, mesh=_MESH, out_type=out_type,
                     scratch_types=scratch_types, compiler_params=_SC_PARAMS)
  def sc_split(h_d, h_g, out_d, out_g, buf):
    c = lax.axis_index("c")
    s = lax.axis_index("s")

    def build(h, out, rpt, nb, cc):
      for j in range(4):
        p = cc * 4 + j
        for k in range(nb):
          r0 = s * rpt + k * SPB
          pltpu.sync_copy(h.at[pl.ds(r0, SPB), pl.ds(p * 16, 16)], buf)
          pltpu.sync_copy(buf, out.at[p, pl.ds(r0, SPB)])

    for cc in range(2):
      @pl.when(c == cc)
      def _():
        build(h_d, out_d, RPT_D, 4, cc)
        build(h_g, out_g, RPT_G, 2, cc)

  return sc_split


def _sc_agg_build():
  f32 = jnp.float32
  out_type = [
      jax.ShapeDtypeStruct((NPD, 128), f32),
      jax.ShapeDtypeStruct((NPD, 128), f32),
      jax.ShapeDtypeStruct((NPG, 128), f32),
      jax.ShapeDtypeStruct((NPG, 128), f32),
  ]
  scratch_types = [
      pltpu.VMEM_SHARED((NPD, 16), f32),
      pltpu.VMEM((TROWS, 128), jnp.int32),
      pltpu.VMEM((TROWS, 128), jnp.int32),
      pltpu.VMEM((NBUF, 128, 16), f32),
      pltpu.VMEM((ZR, 16), f32),
  ] + [pltpu.SemaphoreType.DMA] * NBUF

  @functools.partial(pl.kernel, mesh=_MESH, out_type=out_type,
                     scratch_types=scratch_types, compiler_params=_SC_PARAMS)
  def sc_agg(
             sp_d, sp_g,
             s_dd, d_dd, s_dg, d_dg, s_gd, d_gd, s_gg, d_gg,
             agg_dd, agg_gd, agg_dg, agg_gg,
             acc, srcidx, dstidx, rowbuf, zrow,
             sem0, sem1, sem2):
    c = lax.axis_index("c")
    s = lax.axis_index("s")
    sems = [sem0, sem1, sem2]
    hd = [sp_d.at[p] for p in range(8)]
    hg = [sp_g.at[p] for p in range(8)]

    def zfill(i, _):
      zrow[i] = jnp.zeros((16,), f32)
      return 0
    lax.fori_loop(0, ZR, zfill, 0)

    def load_edges(src2d, dst2d):
      base = s * TROWS
      pltpu.sync_copy(src2d.at[pl.ds(base, TROWS)], srcidx)
      pltpu.sync_copy(dst2d.at[pl.ds(base, TROWS)], dstidx)

    def do_chunk(hsplit, agg_out, p, rpt, nz):
      for k in range(nz):
        pltpu.sync_copy(zrow, acc.at[pl.ds(s * rpt + k * ZR, ZR)])
      plsc.subcore_barrier()
      def fire(g, b):
        pltpu.async_copy(hsplit.at[srcidx.at[g]], rowbuf.at[b], sems[b])
      def drain_scatter(g, b):
        pltpu.make_async_copy(hsplit.at[srcidx.at[g]],
                              rowbuf.at[b], sems[b]).wait()
        pltpu.sync_copy(rowbuf.at[b], acc.at[dstidx.at[g]], add=True)
      for b in range(NBUF):
        fire(b, b)
      def bbody(bi, _):
        for b in range(NBUF):
          g = bi * NBUF + b
          drain_scatter(g, b)
          fire(g + NBUF, b)
        return 0
      lax.fori_loop(0, NBATCH - 1, bbody, 0)
      for b in range(NBUF):
        drain_scatter((NBATCH - 1) * NBUF + b, b)
      plsc.subcore_barrier()
      pltpu.sync_copy(acc.at[pl.ds(s * rpt, rpt)],
                      agg_out.at[pl.ds(s * rpt, rpt), pl.ds(p * 16, 16)])
      plsc.subcore_barrier()

    etys = [
        (s_dd, d_dd, hd, agg_dd, RPT_D, 32),
        (s_gd, d_gd, hg, agg_gd, RPT_D, 32),
        (s_dg, d_dg, hd, agg_dg, RPT_G, 16),
        (s_gg, d_gg, hg, agg_gg, RPT_G, 16),
    ]
    for cc in range(2):
      @pl.when(c == cc)
      def _():
        for (src2d, dst2d, hs, agg_out, rpt, nz) in etys:
          load_edges(src2d, dst2d)
          for j in range(4):
            do_chunk(hs[cc * 4 + j], agg_out, cc * 4 + j, rpt, nz)

  return sc_agg


_sc_counts = _sc_counts_build()
_sc_split = _sc_split_build()
_sc_agg = _sc_agg_build()


def _proj(x, W, b):
  n = x.shape[0]
  def body(x_ref, w_ref, b_ref, o_ref):
    o_ref[:] = (jnp.dot(x_ref[:], w_ref[:], preferred_element_type=jnp.float32,
                  precision=lax.Precision.HIGHEST)
                + b_ref[:][None, :])
  return pl.pallas_call(
      body,
      grid=(n // BM,),
      in_specs=[
          pl.BlockSpec((BM, 128), lambda i: (i, 0)),
          pl.BlockSpec((128, 128), lambda i: (0, 0)),
          pl.BlockSpec((128,), lambda i: (0,)),
      ],
      out_specs=pl.BlockSpec((BM, 128), lambda i: (i, 0)),
      out_shape=jax.ShapeDtypeStruct((n, 128), jnp.float32),
  )(x, W, b)


def _layer_nt(agg_a, cnt_a, Wa, ba, agg_b, cnt_b, Wb, bb, h, Ws, bs):
  n = h.shape[0]
  def body(aa_ref, ab_ref, h_ref, ca_ref, cb_ref,
           wa_ref, wb_ref, ws_ref, ba_ref, bb_ref, bs_ref, o_ref):
    ca = ca_ref[:]
    cb = cb_ref[:]
    ia = 1.0 / jnp.maximum(ca, 1.0)
    ib = 1.0 / jnp.maximum(cb, 1.0)
    ma = (ca > 0.0).astype(jnp.float32)
    mb = (cb > 0.0).astype(jnp.float32)
    xa = aa_ref[:] * ia[:, None]
    xb = ab_ref[:] * ib[:, None]
    na = (jnp.dot(xa, wa_ref[:], preferred_element_type=jnp.float32,
                  precision=lax.Precision.HIGHEST)
          + ma[:, None] * ba_ref[:][None, :])
    nb = (jnp.dot(xb, wb_ref[:], preferred_element_type=jnp.float32,
                  precision=lax.Precision.HIGHEST)
          + mb[:, None] * bb_ref[:][None, :])
    hs = (jnp.dot(h_ref[:], ws_ref[:], preferred_element_type=jnp.float32,
                  precision=lax.Precision.HIGHEST)
          + bs_ref[:][None, :])
    o_ref[:] = jnp.maximum(0.5 * (na + nb) + hs, 0.0)
  mat = lambda: pl.BlockSpec((BM, 128), lambda i: (i, 0))
  vec = lambda: pl.BlockSpec((BM,), lambda i: (i,))
  wsp = lambda: pl.BlockSpec((128, 128), lambda i: (0, 0))
  bsp = lambda: pl.BlockSpec((128,), lambda i: (0,))
  return pl.pallas_call(
      body,
      grid=(n // BM,),
      in_specs=[mat(), mat(), mat(), vec(), vec(),
                wsp(), wsp(), wsp(), bsp(), bsp(), bsp()],
      out_specs=pl.BlockSpec((BM, 128), lambda i: (i, 0)),
      out_shape=jax.ShapeDtypeStruct((n, 128), jnp.float32),
  )(agg_a, agg_b, h, cnt_a, cnt_b, Wa, Wb, Ws, ba, bb, bs)


def kernel(gene_feat, emb_drug, Wf, bf, W_et, b_et, W_self, b_self,
           drug_identity, edge_dd, edge_dg, edge_gd, edge_gg):
  f32 = jnp.float32
  i32 = jnp.int32

  h_d = jnp.pad(emb_drug, ((0, NPD - N_DRUG), (0, 0)))
  gene_p = jnp.pad(gene_feat, ((0, NPG - N_GENE), (0, 0)))

  padn = EP - E
  ar = jnp.arange(padn, dtype=i32)
  pad_src_d = ar % N_DRUG
  pad_src_g = ar % N_GENE
  pad_dst_d = N_DRUG + ar % (NPD - N_DRUG)
  pad_dst_g = N_GENE + ar % (NPG - N_GENE)

  def prep(e, src_is_drug, dst_is_drug):
    src = jnp.concatenate([e[0], pad_src_d if src_is_drug else pad_src_g])
    dst = jnp.concatenate([e[1], pad_dst_d if dst_is_drug else pad_dst_g])
    return src.reshape(EROWS, 128), dst.reshape(EROWS, 128)

  s_dd, d_dd = prep(edge_dd, True, True)
  s_dg, d_dg = prep(edge_dg, True, False)
  s_gd, d_gd = prep(edge_gd, False, True)
  s_gg, d_gg = prep(edge_gg, False, False)

  h_g = _proj(gene_p, Wf, bf)

  cnt_dd, cnt_gd, cnt_dg, cnt_gg = _sc_counts(d_dd, d_gd, d_dg, d_gg)

  h = [h_d, h_g]
  for l in range(L):
    sp_d, sp_g = _sc_split(h[0], h[1])
    agg_dd, agg_gd, agg_dg, agg_gg = _sc_agg(
        sp_d, sp_g,
        s_dd, d_dd, s_dg, d_dg, s_gd, d_gd, s_gg, d_gg)
    new_hd = _layer_nt(agg_dd, cnt_dd, W_et[l, 0], b_et[l, 0],
                       agg_gd, cnt_gd, W_et[l, 2], b_et[l, 2],
                       h[0], W_self[l, 0], b_self[l, 0])
    new_hg = _layer_nt(agg_dg, cnt_dg, W_et[l, 1], b_et[l, 1],
                       agg_gg, cnt_gg, W_et[l, 3], b_et[l, 3],
                       h[1], W_self[l, 1], b_self[l, 1])
    h = [new_hd, new_hg]

  return (h[0][:N_DRUG], h[1][:N_GENE])
